# baseline (speedup 1.0000x reference)
import functools
import numpy as np
import jax
import jax.numpy as jnp
from jax.experimental import pallas as pl
from jax.experimental.pallas import tpu as pltpu

_D = 32
_H = 4
_HD = _D // _H
_N = 5
_PATCH = 16
_CHANS = 3
_IMG = 32
_KFLAT = _CHANS * _IMG * _IMG
_NUM_CLASSES = 10
_OUTPAD = 16
_EPS = 1e-6
_GELU_C = float(np.sqrt(2.0 / np.pi))


def _gelu_tanh(v):
    return 0.5 * v * (1.0 + jnp.tanh(_GELU_C * (v + 0.044715 * v * v * v)))


def _vit_kernel(xt_ref, wbig_ref, embed_ref, s32_ref, r32_ref,
                wqkv_ref, bqkv_ref, projw_ref, projb_ref,
                fc1w_ref, fc1b_ref, fc2w_ref, fc2b_ref,
                headw_ref, headb_ref, o_ref, *, tb, depth):
    lanes = _N * tb

    def bcast(ref, idx, rows, width):
        if idx is None:
            col = ref[...]
        else:
            col = ref[idx]
        return jnp.broadcast_to(col, (rows, width))

    def lnorm(v, width):
        vb = v.astype(jnp.bfloat16)
        mu = jnp.dot(s32_ref[...], vb, preferred_element_type=jnp.float32)
        ms = jnp.dot(s32_ref[...], vb * vb, preferred_element_type=jnp.float32)
        var = ms - mu * mu
        return ((v - mu) * jax.lax.rsqrt(var + _EPS)).astype(jnp.bfloat16)

    xb = xt_ref[...].astype(jnp.bfloat16)
    e = jnp.dot(wbig_ref[...], xb, preferred_element_type=jnp.float32)
    e = e + bcast(embed_ref, None, _N * _D, tb)

    x = jnp.concatenate([e[t * _D:(t + 1) * _D, :] for t in range(_N)],
                        axis=1)

    for l in range(depth):
        hn = lnorm(x, lanes)
        qkv = jnp.dot(wqkv_ref[l], hn,
                      preferred_element_type=jnp.float32) \
            + bcast(bqkv_ref, l, 3 * _D, lanes)
        q = qkv[0:_D, :].astype(jnp.bfloat16)
        k = qkv[_D:2 * _D, :].astype(jnp.bfloat16)
        v = qkv[2 * _D:3 * _D, :]

        ctxs = []
        for a in range(_N):
            qa = q[:, a * tb:(a + 1) * tb]
            logits = [jnp.dot(r32_ref[...], qa * k[:, b * tb:(b + 1) * tb],
                              preferred_element_type=jnp.float32)
                      for b in range(_N)]
            m = logits[0]
            for b in range(1, _N):
                m = jnp.maximum(m, logits[b])
            exps = [jnp.exp(lg - m) for lg in logits]
            z = exps[0]
            for b in range(1, _N):
                z = z + exps[b]
            num = exps[0] * v[:, 0:tb]
            for b in range(1, _N):
                num = num + exps[b] * v[:, b * tb:(b + 1) * tb]
            ctxs.append(num / z)
        ctx = jnp.concatenate(ctxs, axis=1).astype(jnp.bfloat16)
        x = x + jnp.dot(projw_ref[l], ctx,
                        preferred_element_type=jnp.float32) \
            + bcast(projb_ref, l, _D, lanes)

        hn2 = lnorm(x, lanes)
        hm = jnp.dot(fc1w_ref[l], hn2,
                     preferred_element_type=jnp.float32) \
            + bcast(fc1b_ref, l, 4 * _D, lanes)
        hm = _gelu_tanh(hm.astype(jnp.bfloat16))
        x = x + jnp.dot(fc2w_ref[l], hm,
                        preferred_element_type=jnp.float32) \
            + bcast(fc2b_ref, l, _D, lanes)

    cn = lnorm(x[:, 0:tb], tb)
    o_ref[...] = jnp.dot(headw_ref[...], cn,
                         preferred_element_type=jnp.float32) \
        + bcast(headb_ref, None, _OUTPAD, tb)


def kernel(x, patch_w, vec32, vec128, wqkv, bqkv, attn_mask, proj_w,
           fc1_w, fc2_w, head_w):
    del attn_mask
    B = x.shape[0]
    L = wqkv.shape[0]
    scale = float(_HD) ** -0.5
    bf = jnp.bfloat16

    pwt = patch_w.T.reshape(_D, _CHANS, _PATCH, _PATCH)
    blocks = [jnp.zeros((_D, _KFLAT), patch_w.dtype)]
    for ph in range(2):
        for pw in range(2):
            padded = jnp.pad(pwt, ((0, 0), (0, 0),
                                   (ph * _PATCH, _IMG - (ph + 1) * _PATCH),
                                   (pw * _PATCH, _IMG - (pw + 1) * _PATCH)))
            blocks.append(padded.reshape(_D, _KFLAT))
    wbigT = jnp.concatenate(blocks, axis=0).astype(bf)
    embed_col = vec32[0:_N].reshape(_N * _D, 1)

    ln1_w = vec32[_N + 0::6][:L]
    ln1_b = vec32[_N + 1::6][:L]
    proj_b = vec32[_N + 2::6][:L]
    ln2_w = vec32[_N + 3::6][:L]
    ln2_b = vec32[_N + 4::6][:L]
    fc2_b = vec32[_N + 5::6][:L]
    norm_w = vec32[_N + 6 * L]
    norm_b = vec32[_N + 6 * L + 1]

    wq_all = wqkv.reshape(L, _D, _H, 3, _HD).transpose(0, 1, 3, 2, 4)
    wq_all = wq_all.reshape(L, _D, 3 * _D)
    bq_all = bqkv.reshape(L, 1, _H, 3, _HD).transpose(0, 1, 3, 2, 4)
    bq_all = bq_all.reshape(L, 3 * _D)
    smul = jnp.concatenate([jnp.full((_D,), scale),
                            jnp.ones((2 * _D,))]).astype(jnp.float32)
    wq_all = wq_all * smul[None, None, :]
    bq_all = bq_all * smul[None, :]
    wqkvT = (wq_all * ln1_w[:, :, None]).transpose(0, 2, 1).astype(bf)
    bqkvT = (jnp.einsum('lds,ld->ls', wq_all, ln1_b) + bq_all)[..., None]

    projT = proj_w.transpose(0, 2, 1).astype(bf)
    projbT = proj_b[..., None]
    fc1T = (fc1_w * ln2_w[:, :, None]).transpose(0, 2, 1).astype(bf)
    fc1bT = (jnp.einsum('ldm,ld->lm', fc1_w, ln2_b) + vec128[:L])[..., None]
    fc2T = fc2_w.transpose(0, 2, 1).astype(bf)
    fc2bT = fc2_b[..., None]

    hw = head_w[:, :_NUM_CLASSES]
    headT = jnp.zeros((_OUTPAD, _D), jnp.float32)
    headT = headT.at[:_NUM_CLASSES].set((hw * norm_w[:, None]).T).astype(bf)
    headb = jnp.zeros((_OUTPAD,), jnp.float32)
    headb = (headb.at[:_NUM_CLASSES]
             .set(norm_b @ hw + vec128[L, :_NUM_CLASSES]))[:, None]

    r32 = jnp.asarray(np.kron(np.eye(_H), np.ones((_HD, _HD))), dtype=bf)
    s32 = jnp.asarray(np.ones((_D, _D)) / _D, dtype=bf)

    xt = x.reshape(B, _KFLAT).T

    tb = 1024
    while B % tb:
        tb //= 2
    grid = (B // tb,)

    weights = [wbigT, embed_col, s32, r32, wqkvT, bqkvT, projT, projbT,
               fc1T, fc1bT, fc2T, fc2bT, headT, headb]

    def fixed(a):
        nd = a.ndim
        return pl.BlockSpec(a.shape, lambda i, _nd=nd: (0,) * _nd)

    kern = functools.partial(_vit_kernel, tb=tb, depth=L)
    out = pl.pallas_call(
        kern,
        grid=grid,
        out_shape=jax.ShapeDtypeStruct((_OUTPAD, B), jnp.float32),
        in_specs=[pl.BlockSpec((_KFLAT, tb), lambda i: (0, i))] +
                 [fixed(a) for a in weights],
        out_specs=pl.BlockSpec((_OUTPAD, tb), lambda i: (0, i)),
        compiler_params=pltpu.CompilerParams(
            dimension_semantics=("parallel",)),
    )(xt, *weights)
    return out[:_NUM_CLASSES].T


# speedup vs baseline: 1.1317x; 1.1317x over previous
import functools
import numpy as np
import jax
import jax.numpy as jnp
from jax.experimental import pallas as pl
from jax.experimental.pallas import tpu as pltpu

_D = 32
_H = 4
_HD = _D // _H
_N = 5
_PATCH = 16
_CHANS = 3
_IMG = 32
_KFLAT = _CHANS * _IMG * _IMG
_NUM_CLASSES = 10
_OUTPAD = 16
_EPS = 1e-6
_GELU_C = float(np.sqrt(2.0 / np.pi))


def _gelu_tanh(v):
    return 0.5 * v * (1.0 + jnp.tanh(_GELU_C * (v + 0.044715 * v * v * v)))


def _vit_kernel(xt_ref, wbig_ref, embed_ref, s32_ref, r32_ref,
                wqkv_ref, bqkv_ref, projw_ref, projb_ref,
                fc1w_ref, fc1b_ref, fc2w_ref, fc2b_ref,
                headw_ref, headb_ref, o_ref, *, tb, depth):
    lanes = _N * tb

    def bcast(ref, idx, rows, width):
        if idx is None:
            col = ref[...]
        else:
            col = ref[idx]
        return jnp.broadcast_to(col, (rows, width))

    def lnorm(v, width):
        vb = v.astype(jnp.bfloat16)
        mu = jnp.dot(s32_ref[...], vb, preferred_element_type=jnp.float32)
        ms = jnp.dot(s32_ref[...], vb * vb, preferred_element_type=jnp.float32)
        var = ms - mu * mu
        return ((v - mu) * jax.lax.rsqrt(var + _EPS)).astype(jnp.bfloat16)

    xb = xt_ref[...].astype(jnp.bfloat16)
    e = jnp.dot(wbig_ref[...], xb, preferred_element_type=jnp.float32)
    e = e + bcast(embed_ref, None, _N * _D, tb)

    x = jnp.concatenate([e[t * _D:(t + 1) * _D, :] for t in range(_N)],
                        axis=1)

    for l in range(depth):
        hn = lnorm(x, lanes)
        qkv = jnp.dot(wqkv_ref[l], hn,
                      preferred_element_type=jnp.float32) \
            + bcast(bqkv_ref, l, 3 * _D, lanes)
        q = qkv[0:_D, :].astype(jnp.bfloat16)
        k = qkv[_D:2 * _D, :].astype(jnp.bfloat16)
        v = qkv[2 * _D:3 * _D, :]

        ctxs = []
        for a in range(_N):
            qa = q[:, a * tb:(a + 1) * tb]
            logits = [jnp.dot(r32_ref[...], qa * k[:, b * tb:(b + 1) * tb],
                              preferred_element_type=jnp.float32)
                      for b in range(_N)]
            m = logits[0]
            for b in range(1, _N):
                m = jnp.maximum(m, logits[b])
            exps = [jnp.exp(lg - m) for lg in logits]
            z = exps[0]
            for b in range(1, _N):
                z = z + exps[b]
            num = exps[0] * v[:, 0:tb]
            for b in range(1, _N):
                num = num + exps[b] * v[:, b * tb:(b + 1) * tb]
            ctxs.append(num / z)
        ctx = jnp.concatenate(ctxs, axis=1).astype(jnp.bfloat16)
        x = x + jnp.dot(projw_ref[l], ctx,
                        preferred_element_type=jnp.float32) \
            + bcast(projb_ref, l, _D, lanes)

        hn2 = lnorm(x, lanes)
        hm = jnp.dot(fc1w_ref[l], hn2,
                     preferred_element_type=jnp.float32) \
            + bcast(fc1b_ref, l, 4 * _D, lanes)
        hm = _gelu_tanh(hm.astype(jnp.bfloat16))
        x = x + jnp.dot(fc2w_ref[l], hm,
                        preferred_element_type=jnp.float32) \
            + bcast(fc2b_ref, l, _D, lanes)

    cn = lnorm(x[:, 0:tb], tb)
    o_ref[...] = jnp.dot(headw_ref[...], cn,
                         preferred_element_type=jnp.float32) \
        + bcast(headb_ref, None, _OUTPAD, tb)


def kernel(x, patch_w, vec32, vec128, wqkv, bqkv, attn_mask, proj_w,
           fc1_w, fc2_w, head_w):
    del attn_mask
    B = x.shape[0]
    L = wqkv.shape[0]
    scale = float(_HD) ** -0.5
    bf = jnp.bfloat16

    eye2 = jnp.eye(2, dtype=patch_w.dtype)
    w4 = patch_w.reshape(_CHANS, _PATCH, _PATCH, _D)
    wbig = jnp.einsum('cabe,hH,wV->HVechawb', w4, eye2, eye2)
    wbig = wbig.reshape((_N - 1) * _D, _KFLAT)
    wbigT = jnp.concatenate(
        [jnp.zeros((_D, _KFLAT), patch_w.dtype), wbig], axis=0).astype(bf)
    embed_col = vec32[0:_N].reshape(_N * _D, 1)

    ln1_w = vec32[_N + 0::6][:L]
    ln1_b = vec32[_N + 1::6][:L]
    proj_b = vec32[_N + 2::6][:L]
    ln2_w = vec32[_N + 3::6][:L]
    ln2_b = vec32[_N + 4::6][:L]
    fc2_b = vec32[_N + 5::6][:L]
    norm_w = vec32[_N + 6 * L]
    norm_b = vec32[_N + 6 * L + 1]

    wq_all = wqkv.reshape(L, _D, _H, 3, _HD).transpose(0, 1, 3, 2, 4)
    wq_all = wq_all.reshape(L, _D, 3 * _D)
    bq_all = bqkv.reshape(L, 1, _H, 3, _HD).transpose(0, 1, 3, 2, 4)
    bq_all = bq_all.reshape(L, 3 * _D)
    smul = jnp.concatenate([jnp.full((_D,), scale),
                            jnp.ones((2 * _D,))]).astype(jnp.float32)
    wq_all = wq_all * smul[None, None, :]
    bq_all = bq_all * smul[None, :]
    wqkvT = (wq_all * ln1_w[:, :, None]).transpose(0, 2, 1).astype(bf)
    bqkvT = (jnp.einsum('lds,ld->ls', wq_all, ln1_b) + bq_all)[..., None]

    projT = proj_w.transpose(0, 2, 1).astype(bf)
    projbT = proj_b[..., None]
    fc1T = (fc1_w * ln2_w[:, :, None]).transpose(0, 2, 1).astype(bf)
    fc1bT = (jnp.einsum('ldm,ld->lm', fc1_w, ln2_b) + vec128[:L])[..., None]
    fc2T = fc2_w.transpose(0, 2, 1).astype(bf)
    fc2bT = fc2_b[..., None]

    hw = head_w[:, :_NUM_CLASSES]
    headT = jnp.zeros((_OUTPAD, _D), jnp.float32)
    headT = headT.at[:_NUM_CLASSES].set((hw * norm_w[:, None]).T).astype(bf)
    headb = jnp.zeros((_OUTPAD,), jnp.float32)
    headb = (headb.at[:_NUM_CLASSES]
             .set(norm_b @ hw + vec128[L, :_NUM_CLASSES]))[:, None]

    r32 = jnp.asarray(np.kron(np.eye(_H), np.ones((_HD, _HD))), dtype=bf)
    s32 = jnp.asarray(np.ones((_D, _D)) / _D, dtype=bf)

    xt = x.reshape(B, _KFLAT).T

    tb = 1024
    while B % tb:
        tb //= 2
    grid = (B // tb,)

    weights = [wbigT, embed_col, s32, r32, wqkvT, bqkvT, projT, projbT,
               fc1T, fc1bT, fc2T, fc2bT, headT, headb]

    def fixed(a):
        nd = a.ndim
        return pl.BlockSpec(a.shape, lambda i, _nd=nd: (0,) * _nd)

    kern = functools.partial(_vit_kernel, tb=tb, depth=L)
    out = pl.pallas_call(
        kern,
        grid=grid,
        out_shape=jax.ShapeDtypeStruct((_OUTPAD, B), jnp.float32),
        in_specs=[pl.BlockSpec((_KFLAT, tb), lambda i: (0, i))] +
                 [fixed(a) for a in weights],
        out_specs=pl.BlockSpec((_OUTPAD, tb), lambda i: (0, i)),
        compiler_params=pltpu.CompilerParams(
            dimension_semantics=("parallel",)),
    )(xt, *weights)
    return out[:_NUM_CLASSES].T
